# baseline (speedup 1.0000x reference)
import functools

import jax
import jax.numpy as jnp
from jax import lax
from jax.experimental import pallas as pl
from jax.experimental.pallas import tpu as pltpu
from jax.experimental.pallas import tpu_sc as plsc

B = 8192
DH = 1024
D = 2048
E = 8
BT = 512
BM = 256
BN = 1024
M_PAD = B * 2 + E * BM
M_TILES = M_PAD // BM
N_TILES = D // BN
NW = 32
TPW = B // NW


def _gate_body(x_ref, y_ref, gw_ref, gb_ref, lg_ref, xb_ref, yb_ref):
    xv = x_ref[...]
    yv = y_ref[...]
    dnums = (((1,), (1,)), ((), ()))
    inpb = jnp.concatenate([xv, yv], axis=1).astype(jnp.bfloat16)
    gwb = gw_ref[...].astype(jnp.bfloat16)
    lg_ref[...] = (
        lax.dot_general(inpb, gwb, dnums, preferred_element_type=jnp.float32)
        + gb_ref[...]
    )
    xb_ref[...] = xv.astype(jnp.bfloat16)
    yb_ref[...] = yv.astype(jnp.bfloat16)


def _route_body(lg_ref, p0_ref, p1_ref, g0_ref, g1_ref, te_ref):
    lg = lg_ref[...]
    idx8 = lax.broadcasted_iota(jnp.int32, (B, E), 1)
    m1 = jnp.max(lg, axis=1, keepdims=True)
    i1 = jnp.min(jnp.where(lg == m1, idx8, E), axis=1, keepdims=True)
    l2 = jnp.where(idx8 == i1, -1e30, lg)
    m2 = jnp.max(l2, axis=1, keepdims=True)
    i2 = jnp.min(jnp.where(l2 == m2, idx8, E), axis=1, keepdims=True)
    g0 = 1.0 / (1.0 + jnp.exp(m2 - m1))
    g1 = 1.0 - g0
    oh1 = (idx8 == i1).astype(jnp.float32)
    oh2 = (idx8 == i2).astype(jnp.float32)
    h = oh1 + oh2

    CH = 512
    tri = (
        lax.broadcasted_iota(jnp.int32, (CH, CH), 0)
        > lax.broadcasted_iota(jnp.int32, (CH, CH), 1)
    ).astype(jnp.float32)
    dn = (((1,), (0,)), ((), ()))
    carry = jnp.zeros((1, E), jnp.float32)
    chunks = []
    for c in range(B // CH):
        hc = lax.slice(h, (c * CH, 0), ((c + 1) * CH, E))
        chunks.append(carry + lax.dot_general(tri, hc, dn,
                                              preferred_element_type=jnp.float32))
        carry = carry + jnp.sum(hc, axis=0, keepdims=True)
    hexc = jnp.concatenate(chunks, axis=0)
    counts = carry.astype(jnp.int32)

    ntiles = (counts + (BM - 1)) // BM
    tri8 = (
        lax.broadcasted_iota(jnp.int32, (E, E), 0)
        < lax.broadcasted_iota(jnp.int32, (E, E), 1)
    ).astype(jnp.float32)
    off_t = lax.dot_general(ntiles.astype(jnp.float32), tri8, dn,
                            preferred_element_type=jnp.float32)
    off_rows = off_t * BM

    posf = hexc + off_rows
    p0_ref[...] = jnp.sum(oh1 * posf, axis=1, keepdims=True).astype(jnp.int32)
    p1_ref[...] = jnp.sum(oh2 * posf, axis=1, keepdims=True).astype(jnp.int32)
    g0_ref[...] = jnp.broadcast_to(g0, (B, 16))
    g1_ref[...] = jnp.broadcast_to(g1, (B, 16))

    off_ti = off_t.astype(jnp.int32)
    mti = lax.broadcasted_iota(jnp.int32, (1, 128), 1)
    te = jnp.zeros((1, 128), jnp.int32)
    for e in range(E):
        te = te + (mti >= lax.slice(off_ti, (0, e), (1, e + 1))).astype(jnp.int32)
    te = jnp.clip(te - 1, 0, E - 1)
    te_ref[...] = jnp.broadcast_to(te, (8, 128))


def _dispatch_body(xb_hbm, yb_hbm, p0_hbm, p1_hbm, xa_hbm, xc_hbm,
                   i0_v, i1_v, xv, yv, sem):
    wid = lax.axis_index("s") * 2 + lax.axis_index("c")
    CH = 32
    def chunk(c, _):
        base = wid * TPW + c * CH
        pltpu.sync_copy(p0_hbm.at[pl.ds(base, CH)], i0_v)
        pltpu.sync_copy(p1_hbm.at[pl.ds(base, CH)], i1_v)
        pltpu.sync_copy(xb_hbm.at[pl.ds(base, CH)], xv)
        pltpu.sync_copy(yb_hbm.at[pl.ds(base, CH)], yv)
        pltpu.async_copy(xv, xa_hbm.at[i0_v], sem)
        pltpu.async_copy(xv, xa_hbm.at[i1_v], sem)
        pltpu.async_copy(yv, xc_hbm.at[i0_v], sem)
        cp = pltpu.async_copy(yv, xc_hbm.at[i1_v], sem)
        cp.wait()
        cp.wait()
        cp.wait()
        cp.wait()
        return 0
    lax.fori_loop(0, TPW // CH, chunk, 0)


def _gmm_body(te_ref, xa_ref, xc_ref, w_ref, b_ref, y_ref):
    dnums = (((1,), (1,)), ((), ()))
    wa = w_ref[0, :, :DH]
    wb = w_ref[0, :, DH:]
    acc = (
        lax.dot_general(xa_ref[...], wa, dnums,
                        preferred_element_type=jnp.float32)
        + lax.dot_general(xc_ref[...], wb, dnums,
                          preferred_element_type=jnp.float32)
    )
    y_ref[...] = acc + b_ref[0]


def _combine_body(y_hbm, p0_hbm, p1_hbm, g0_hbm, out_hbm,
                  i0_v, i1_v, gg0, r0, r1, s0, s1):
    wid = lax.axis_index("s") * 2 + lax.axis_index("c")
    wbase = wid * TPW
    pltpu.sync_copy(p0_hbm.at[pl.ds(wbase, TPW)], i0_v)
    pltpu.sync_copy(p1_hbm.at[pl.ds(wbase, TPW)], i1_v)
    pltpu.sync_copy(g0_hbm.at[pl.ds(wbase, TPW)], gg0)
    TCH = 16
    def chunk(c, _):
        idx0 = i0_v[pl.ds(c * TCH, TCH)]
        idx1 = i1_v[pl.ds(c * TCH, TCH)]
        cp0 = pltpu.async_copy(y_hbm.at[idx0], r0, s0)
        cp1 = pltpu.async_copy(y_hbm.at[idx1], r1, s1)
        cp0.wait()
        cp1.wait()

        def tok(i, _):
            g0s = gg0[c * TCH + i, :]
            g1s = 1.0 - g0s

            def col(j, _):
                r0[i, pl.ds(j * 16, 16)] = (
                    g0s * r0[i, pl.ds(j * 16, 16)]
                    + g1s * r1[i, pl.ds(j * 16, 16)]
                )
                return 0
            lax.fori_loop(0, D // 16, col, 0)
            return 0
        lax.fori_loop(0, TCH, tok, 0)
        pltpu.sync_copy(r0, out_hbm.at[pl.ds(wbase + c * TCH, TCH)])
        return 0
    lax.fori_loop(0, TPW // TCH, chunk, 0)


def kernel(x, y, W_experts, b_experts, gate_W, gate_b):
    Wb = W_experts.astype(jnp.bfloat16)
    b3 = b_experts.reshape(E, 1, D)
    gb2 = gate_b.reshape(1, E)

    lg, xb, yb = pl.pallas_call(
        _gate_body,
        grid=(B // BT,),
        in_specs=[
            pl.BlockSpec((BT, DH), lambda t: (t, 0)),
            pl.BlockSpec((BT, DH), lambda t: (t, 0)),
            pl.BlockSpec((E, D), lambda t: (0, 0)),
            pl.BlockSpec((1, E), lambda t: (0, 0)),
        ],
        out_specs=[
            pl.BlockSpec((BT, E), lambda t: (t, 0)),
            pl.BlockSpec((BT, DH), lambda t: (t, 0)),
            pl.BlockSpec((BT, DH), lambda t: (t, 0)),
        ],
        out_shape=[
            jax.ShapeDtypeStruct((B, E), jnp.float32),
            jax.ShapeDtypeStruct((B, DH), jnp.bfloat16),
            jax.ShapeDtypeStruct((B, DH), jnp.bfloat16),
        ],
    )(x, y, gate_W, gb2)

    p0, p1, g0, g1, te = pl.pallas_call(
        _route_body,
        out_shape=[
            jax.ShapeDtypeStruct((B, 1), jnp.int32),
            jax.ShapeDtypeStruct((B, 1), jnp.int32),
            jax.ShapeDtypeStruct((B, 16), jnp.float32),
            jax.ShapeDtypeStruct((B, 16), jnp.float32),
            jax.ShapeDtypeStruct((8, 128), jnp.int32),
        ],
    )(lg)
    p0f = p0.reshape(B)
    p1f = p1.reshape(B)
    te_row = te[0]

    xbi = lax.bitcast_convert_type(xb.reshape(B, DH // 2, 2), jnp.int32)
    ybi = lax.bitcast_convert_type(yb.reshape(B, DH // 2, 2), jnp.int32)

    mesh = plsc.VectorSubcoreMesh(core_axis_name="c", subcore_axis_name="s")
    xa_i, xc_i = pl.kernel(
        _dispatch_body,
        out_type=[
            jax.ShapeDtypeStruct((M_PAD, DH // 2), jnp.int32),
            jax.ShapeDtypeStruct((M_PAD, DH // 2), jnp.int32),
        ],
        mesh=mesh,
        scratch_types=[
            pltpu.VMEM((32,), jnp.int32),
            pltpu.VMEM((32,), jnp.int32),
            pltpu.VMEM((32, DH // 2), jnp.int32),
            pltpu.VMEM((32, DH // 2), jnp.int32),
            pltpu.SemaphoreType.DMA,
        ],
    )(xbi, ybi, p0f, p1f)
    xa = lax.bitcast_convert_type(xa_i, jnp.bfloat16).reshape(M_PAD, DH)
    xc = lax.bitcast_convert_type(xc_i, jnp.bfloat16).reshape(M_PAD, DH)

    ysort = pl.pallas_call(
        _gmm_body,
        grid_spec=pltpu.PrefetchScalarGridSpec(
            num_scalar_prefetch=1,
            grid=(N_TILES, M_TILES),
            in_specs=[
                pl.BlockSpec((BM, DH), lambda n, m, te_r: (m, 0)),
                pl.BlockSpec((BM, DH), lambda n, m, te_r: (m, 0)),
                pl.BlockSpec((1, BN, D), lambda n, m, te_r: (te_r[m], n, 0)),
                pl.BlockSpec((1, 1, BN), lambda n, m, te_r: (te_r[m], 0, n)),
            ],
            out_specs=pl.BlockSpec((BM, BN), lambda n, m, te_r: (m, n)),
        ),
        out_shape=jax.ShapeDtypeStruct((M_PAD, D), jnp.float32),
        compiler_params=pltpu.CompilerParams(
            dimension_semantics=("arbitrary", "arbitrary"),
        ),
    )(te_row, xa, xc, Wb, b3)

    out = pl.kernel(
        _combine_body,
        out_type=jax.ShapeDtypeStruct((B, D), jnp.float32),
        mesh=mesh,
        scratch_types=[
            pltpu.VMEM((TPW,), jnp.int32),
            pltpu.VMEM((TPW,), jnp.int32),
            pltpu.VMEM((TPW, 16), jnp.float32),
            pltpu.VMEM((16, D), jnp.float32),
            pltpu.VMEM((16, D), jnp.float32),
            pltpu.SemaphoreType.DMA,
            pltpu.SemaphoreType.DMA,
        ],
    )(ysort, p0f, p1f, g0)
    return out


# speedup vs baseline: 2.7393x; 2.7393x over previous
import functools

import jax
import jax.numpy as jnp
from jax import lax
from jax.experimental import pallas as pl
from jax.experimental.pallas import tpu as pltpu
from jax.experimental.pallas import tpu_sc as plsc

B = 8192
DH = 1024
D = 2048
E = 8
BT = 512
BM = 256
BN = 1024
M_PAD = B * 2 + E * BM
M_TILES = M_PAD // BM
N_TILES = D // BN
NW = 32
TPW = B // NW


def _gate_body(x_ref, y_ref, gw_ref, gb_ref, lg_ref):
    xv = x_ref[...]
    yv = y_ref[...]
    dnums = (((1,), (1,)), ((), ()))
    inpb = jnp.concatenate([xv, yv], axis=1).astype(jnp.bfloat16)
    gwb = gw_ref[...].astype(jnp.bfloat16)
    lg_ref[...] = (
        lax.dot_general(inpb, gwb, dnums, preferred_element_type=jnp.float32)
        + gb_ref[...]
    )


def _route_body(lg_ref, p0_ref, p1_ref, g0_ref, g1_ref, te_ref):
    lg = lg_ref[...]
    idx8 = lax.broadcasted_iota(jnp.int32, (B, E), 1)
    m1 = jnp.max(lg, axis=1, keepdims=True)
    i1 = jnp.min(jnp.where(lg == m1, idx8, E), axis=1, keepdims=True)
    l2 = jnp.where(idx8 == i1, -1e30, lg)
    m2 = jnp.max(l2, axis=1, keepdims=True)
    i2 = jnp.min(jnp.where(l2 == m2, idx8, E), axis=1, keepdims=True)
    g0 = 1.0 / (1.0 + jnp.exp(m2 - m1))
    g1 = 1.0 - g0
    oh1 = (idx8 == i1).astype(jnp.float32)
    oh2 = (idx8 == i2).astype(jnp.float32)
    h = oh1 + oh2

    CH = 512
    tri = (
        lax.broadcasted_iota(jnp.int32, (CH, CH), 0)
        > lax.broadcasted_iota(jnp.int32, (CH, CH), 1)
    ).astype(jnp.float32)
    dn = (((1,), (0,)), ((), ()))
    carry = jnp.zeros((1, E), jnp.float32)
    chunks = []
    for c in range(B // CH):
        hc = lax.slice(h, (c * CH, 0), ((c + 1) * CH, E))
        chunks.append(carry + lax.dot_general(tri, hc, dn,
                                              preferred_element_type=jnp.float32))
        carry = carry + jnp.sum(hc, axis=0, keepdims=True)
    hexc = jnp.concatenate(chunks, axis=0)
    counts = carry.astype(jnp.int32)

    ntiles = (counts + (BM - 1)) // BM
    tri8 = (
        lax.broadcasted_iota(jnp.int32, (E, E), 0)
        < lax.broadcasted_iota(jnp.int32, (E, E), 1)
    ).astype(jnp.float32)
    off_t = lax.dot_general(ntiles.astype(jnp.float32), tri8, dn,
                            preferred_element_type=jnp.float32)
    off_rows = off_t * BM

    posf = hexc + off_rows
    p0_ref[...] = jnp.sum(oh1 * posf, axis=1, keepdims=True).astype(jnp.int32)
    p1_ref[...] = jnp.sum(oh2 * posf, axis=1, keepdims=True).astype(jnp.int32)
    g0_ref[...] = jnp.broadcast_to(g0, (B, 16))
    g1_ref[...] = jnp.broadcast_to(g1, (B, 16))

    off_ti = off_t.astype(jnp.int32)
    mti = lax.broadcasted_iota(jnp.int32, (1, 128), 1)
    te = jnp.zeros((1, 128), jnp.int32)
    for e in range(E):
        te = te + (mti >= lax.slice(off_ti, (0, e), (1, e + 1))).astype(jnp.int32)
    te = jnp.clip(te - 1, 0, E - 1)
    te_ref[...] = jnp.broadcast_to(te, (8, 128))


def _dispatch_body(xb_hbm, yb_hbm, p0_hbm, p1_hbm, xa_hbm, xc_hbm,
                   i0_v, i1_v, xv, yv, sem):
    wid = lax.axis_index("s") * 2 + lax.axis_index("c")
    CH = 32
    def chunk(c, _):
        base = wid * TPW + c * CH
        pltpu.sync_copy(p0_hbm.at[pl.ds(base, CH)], i0_v)
        pltpu.sync_copy(p1_hbm.at[pl.ds(base, CH)], i1_v)
        pltpu.sync_copy(xb_hbm.at[pl.ds(base, CH)], xv)
        pltpu.sync_copy(yb_hbm.at[pl.ds(base, CH)], yv)
        pltpu.async_copy(xv, xa_hbm.at[i0_v], sem)
        pltpu.async_copy(xv, xa_hbm.at[i1_v], sem)
        pltpu.async_copy(yv, xc_hbm.at[i0_v], sem)
        cp = pltpu.async_copy(yv, xc_hbm.at[i1_v], sem)
        cp.wait()
        cp.wait()
        cp.wait()
        cp.wait()
        return 0
    lax.fori_loop(0, TPW // CH, chunk, 0)


def _gmm_body(te_ref, xa_ref, xc_ref, w_ref, b_ref, y_ref):
    dnums = (((1,), (1,)), ((), ()))
    wa = w_ref[0, :, :DH]
    wb = w_ref[0, :, DH:]
    acc = (
        lax.dot_general(xa_ref[...].astype(jnp.bfloat16), wa, dnums,
                        preferred_element_type=jnp.float32)
        + lax.dot_general(xc_ref[...].astype(jnp.bfloat16), wb, dnums,
                          preferred_element_type=jnp.float32)
    )
    y_ref[...] = acc + b_ref[0]


def _combine_body(y_hbm, p0_hbm, p1_hbm, g0_hbm, out_hbm,
                  i0_v, i1_v, gg0, r0, r1, s0, s1):
    wid = lax.axis_index("s") * 2 + lax.axis_index("c")
    wbase = wid * TPW
    pltpu.sync_copy(p0_hbm.at[pl.ds(wbase, TPW)], i0_v)
    pltpu.sync_copy(p1_hbm.at[pl.ds(wbase, TPW)], i1_v)
    pltpu.sync_copy(g0_hbm.at[pl.ds(wbase, TPW)], gg0)
    TCH = 16
    def chunk(c, _):
        idx0 = i0_v[pl.ds(c * TCH, TCH)]
        idx1 = i1_v[pl.ds(c * TCH, TCH)]
        cp0 = pltpu.async_copy(y_hbm.at[idx0], r0, s0)
        cp1 = pltpu.async_copy(y_hbm.at[idx1], r1, s1)
        cp0.wait()
        cp1.wait()

        def tok(i, _):
            g0s = gg0[c * TCH + i, :]
            g1s = 1.0 - g0s
            for j in range(D // 16):
                r0[i, pl.ds(j * 16, 16)] = (
                    g0s * r0[i, pl.ds(j * 16, 16)]
                    + g1s * r1[i, pl.ds(j * 16, 16)]
                )
            return 0
        lax.fori_loop(0, TCH, tok, 0)
        pltpu.sync_copy(r0, out_hbm.at[pl.ds(wbase + c * TCH, TCH)])
        return 0
    lax.fori_loop(0, TPW // TCH, chunk, 0)


def kernel(x, y, W_experts, b_experts, gate_W, gate_b):
    Wb = W_experts.astype(jnp.bfloat16)
    b3 = b_experts.reshape(E, 1, D)
    gb2 = gate_b.reshape(1, E)

    lg = pl.pallas_call(
        _gate_body,
        grid=(B // BT,),
        in_specs=[
            pl.BlockSpec((BT, DH), lambda t: (t, 0)),
            pl.BlockSpec((BT, DH), lambda t: (t, 0)),
            pl.BlockSpec((E, D), lambda t: (0, 0)),
            pl.BlockSpec((1, E), lambda t: (0, 0)),
        ],
        out_specs=pl.BlockSpec((BT, E), lambda t: (t, 0)),
        out_shape=jax.ShapeDtypeStruct((B, E), jnp.float32),
    )(x, y, gate_W, gb2)

    p0, p1, g0, g1, te = pl.pallas_call(
        _route_body,
        out_shape=[
            jax.ShapeDtypeStruct((B, 1), jnp.int32),
            jax.ShapeDtypeStruct((B, 1), jnp.int32),
            jax.ShapeDtypeStruct((B, 16), jnp.float32),
            jax.ShapeDtypeStruct((B, 16), jnp.float32),
            jax.ShapeDtypeStruct((8, 128), jnp.int32),
        ],
    )(lg)
    p0f = p0.reshape(B)
    p1f = p1.reshape(B)
    te_row = te[0]

    mesh = plsc.VectorSubcoreMesh(core_axis_name="c", subcore_axis_name="s")
    xa, xc = pl.kernel(
        _dispatch_body,
        out_type=[
            jax.ShapeDtypeStruct((M_PAD, DH), jnp.float32),
            jax.ShapeDtypeStruct((M_PAD, DH), jnp.float32),
        ],
        mesh=mesh,
        scratch_types=[
            pltpu.VMEM((32,), jnp.int32),
            pltpu.VMEM((32,), jnp.int32),
            pltpu.VMEM((32, DH), jnp.float32),
            pltpu.VMEM((32, DH), jnp.float32),
            pltpu.SemaphoreType.DMA,
        ],
    )(x, y, p0f, p1f)

    ysort = pl.pallas_call(
        _gmm_body,
        grid_spec=pltpu.PrefetchScalarGridSpec(
            num_scalar_prefetch=1,
            grid=(N_TILES, M_TILES),
            in_specs=[
                pl.BlockSpec((BM, DH), lambda n, m, te_r: (m, 0)),
                pl.BlockSpec((BM, DH), lambda n, m, te_r: (m, 0)),
                pl.BlockSpec((1, BN, D), lambda n, m, te_r: (te_r[m], n, 0)),
                pl.BlockSpec((1, 1, BN), lambda n, m, te_r: (te_r[m], 0, n)),
            ],
            out_specs=pl.BlockSpec((BM, BN), lambda n, m, te_r: (m, n)),
        ),
        out_shape=jax.ShapeDtypeStruct((M_PAD, D), jnp.float32),
        compiler_params=pltpu.CompilerParams(
            dimension_semantics=("arbitrary", "arbitrary"),
        ),
    )(te_row, xa, xc, Wb, b3)

    out = pl.kernel(
        _combine_body,
        out_type=jax.ShapeDtypeStruct((B, D), jnp.float32),
        mesh=mesh,
        scratch_types=[
            pltpu.VMEM((TPW,), jnp.int32),
            pltpu.VMEM((TPW,), jnp.int32),
            pltpu.VMEM((TPW, 16), jnp.float32),
            pltpu.VMEM((16, D), jnp.float32),
            pltpu.VMEM((16, D), jnp.float32),
            pltpu.SemaphoreType.DMA,
            pltpu.SemaphoreType.DMA,
        ],
    )(ysort, p0f, p1f, g0)
    return out


# speedup vs baseline: 3.3687x; 1.2298x over previous
import functools

import jax
import jax.numpy as jnp
from jax import lax
from jax.experimental import pallas as pl
from jax.experimental.pallas import tpu as pltpu
from jax.experimental.pallas import tpu_sc as plsc

B = 8192
DH = 1024
D = 2048
E = 8
BT = 512
BM = 256
BN = 2048
M_PAD = B * 2 + E * BM
M_TILES = M_PAD // BM
N_TILES = D // BN
NW = 32
TPW = B // NW


def _gate_body(x_ref, y_ref, gw_ref, gb_ref, lg_ref):
    xv = x_ref[...]
    yv = y_ref[...]
    dnums = (((1,), (1,)), ((), ()))
    inpb = jnp.concatenate([xv, yv], axis=1).astype(jnp.bfloat16)
    gwb = gw_ref[...].astype(jnp.bfloat16)
    lg_ref[...] = (
        lax.dot_general(inpb, gwb, dnums, preferred_element_type=jnp.float32)
        + gb_ref[...]
    )


def _route_body(lg_ref, p0_ref, p1_ref, g0_ref, g1_ref, te_ref):
    lg = lg_ref[...]
    idx8 = lax.broadcasted_iota(jnp.int32, (B, E), 1)
    m1 = jnp.max(lg, axis=1, keepdims=True)
    i1 = jnp.min(jnp.where(lg == m1, idx8, E), axis=1, keepdims=True)
    l2 = jnp.where(idx8 == i1, -1e30, lg)
    m2 = jnp.max(l2, axis=1, keepdims=True)
    i2 = jnp.min(jnp.where(l2 == m2, idx8, E), axis=1, keepdims=True)
    g0 = 1.0 / (1.0 + jnp.exp(m2 - m1))
    g1 = 1.0 - g0
    oh1 = (idx8 == i1).astype(jnp.float32)
    oh2 = (idx8 == i2).astype(jnp.float32)
    h = oh1 + oh2

    CH = 512
    tri = (
        lax.broadcasted_iota(jnp.int32, (CH, CH), 0)
        > lax.broadcasted_iota(jnp.int32, (CH, CH), 1)
    ).astype(jnp.float32)
    dn = (((1,), (0,)), ((), ()))
    carry = jnp.zeros((1, E), jnp.float32)
    chunks = []
    for c in range(B // CH):
        hc = lax.slice(h, (c * CH, 0), ((c + 1) * CH, E))
        chunks.append(carry + lax.dot_general(tri, hc, dn,
                                              preferred_element_type=jnp.float32))
        carry = carry + jnp.sum(hc, axis=0, keepdims=True)
    hexc = jnp.concatenate(chunks, axis=0)
    counts = carry.astype(jnp.int32)

    ntiles = (counts + (BM - 1)) // BM
    tri8 = (
        lax.broadcasted_iota(jnp.int32, (E, E), 0)
        < lax.broadcasted_iota(jnp.int32, (E, E), 1)
    ).astype(jnp.float32)
    off_t = lax.dot_general(ntiles.astype(jnp.float32), tri8, dn,
                            preferred_element_type=jnp.float32)
    off_rows = off_t * BM

    posf = hexc + off_rows
    p0_ref[...] = jnp.sum(oh1 * posf, axis=1, keepdims=True).astype(jnp.int32)
    p1_ref[...] = jnp.sum(oh2 * posf, axis=1, keepdims=True).astype(jnp.int32)
    g0_ref[...] = jnp.broadcast_to(g0, (B, 16))
    g1_ref[...] = jnp.broadcast_to(g1, (B, 16))

    off_ti = off_t.astype(jnp.int32)
    mti = lax.broadcasted_iota(jnp.int32, (1, 128), 1)
    te = jnp.zeros((1, 128), jnp.int32)
    for e in range(E):
        te = te + (mti >= lax.slice(off_ti, (0, e), (1, e + 1))).astype(jnp.int32)
    te = jnp.clip(te - 1, 0, E - 1)
    te_ref[...] = jnp.broadcast_to(te, (8, 128))


def _dispatch_body(xb_hbm, yb_hbm, p0_hbm, p1_hbm, xa_hbm, xc_hbm,
                   i0_v, i1_v, xva, yva, xvb, yvb, sla, slb, ssc):
    wid = lax.axis_index("s") * 2 + lax.axis_index("c")
    CH = 16
    NCH = TPW // CH

    def load(c, xv, yv, sem):
        base = wid * TPW + c * CH
        pltpu.async_copy(xb_hbm.at[pl.ds(base, CH)], xv, sem)
        pltpu.async_copy(yb_hbm.at[pl.ds(base, CH)], yv, sem)

    def scat(c, xv, yv, sem):
        base = wid * TPW + c * CH
        pltpu.sync_copy(p0_hbm.at[pl.ds(base, CH)], i0_v)
        pltpu.sync_copy(p1_hbm.at[pl.ds(base, CH)], i1_v)
        pltpu.async_copy(xv, xa_hbm.at[i0_v], sem)
        pltpu.async_copy(xv, xa_hbm.at[i1_v], sem)
        pltpu.async_copy(yv, xc_hbm.at[i0_v], sem)
        cp = pltpu.async_copy(yv, xc_hbm.at[i1_v], sem)
        cp.wait()
        cp.wait()
        cp.wait()
        cp.wait()

    load(0, xva, yva, sla)

    def chunk(c, _):
        @pl.when(c % 2 == 0)
        def _():
            cp = pltpu.make_async_copy(xb_hbm.at[pl.ds(0, CH)], xva, sla)
            cp.wait()
            cp.wait()
            @pl.when(c + 1 < NCH)
            def _():
                load(c + 1, xvb, yvb, slb)
            scat(c, xva, yva, ssc)

        @pl.when(c % 2 == 1)
        def _():
            cp = pltpu.make_async_copy(xb_hbm.at[pl.ds(0, CH)], xvb, slb)
            cp.wait()
            cp.wait()
            @pl.when(c + 1 < NCH)
            def _():
                load(c + 1, xva, yva, sla)
            scat(c, xvb, yvb, ssc)
        return 0
    lax.fori_loop(0, NCH, chunk, 0)


def _gmm_body(te_ref, xa_ref, xc_ref, w_ref, b_ref, y_ref):
    dnums = (((1,), (1,)), ((), ()))
    wa = w_ref[0, :, :DH]
    wb = w_ref[0, :, DH:]
    acc = (
        lax.dot_general(xa_ref[...].astype(jnp.bfloat16), wa, dnums,
                        preferred_element_type=jnp.float32)
        + lax.dot_general(xc_ref[...].astype(jnp.bfloat16), wb, dnums,
                          preferred_element_type=jnp.float32)
    )
    y_ref[...] = acc + b_ref[0]


def _combine_body(y_hbm, p0_hbm, p1_hbm, g0_hbm, out_hbm,
                  i0_v, i1_v, gg0, a0, a1, b0, b1, sa, sb):
    wid = lax.axis_index("s") * 2 + lax.axis_index("c")
    wbase = wid * TPW
    pltpu.sync_copy(p0_hbm.at[pl.ds(wbase, TPW)], i0_v)
    pltpu.sync_copy(p1_hbm.at[pl.ds(wbase, TPW)], i1_v)
    pltpu.sync_copy(g0_hbm.at[pl.ds(wbase, TPW)], gg0)
    TCH = 8
    NCH = TPW // TCH

    def gather(c, r0, r1, sem):
        pltpu.async_copy(y_hbm.at[i0_v.at[pl.ds(c * TCH, TCH)]], r0, sem)
        pltpu.async_copy(y_hbm.at[i1_v.at[pl.ds(c * TCH, TCH)]], r1, sem)

    def compute(c, r0, r1, sem):
        cp = pltpu.make_async_copy(y_hbm.at[pl.ds(0, TCH)], r0, sem)
        cp.wait()
        cp.wait()

        def tok(i, _):
            g0s = gg0[c * TCH + i, :]
            g1s = 1.0 - g0s
            for j in range(D // 16):
                r0[i, pl.ds(j * 16, 16)] = (
                    g0s * r0[i, pl.ds(j * 16, 16)]
                    + g1s * r1[i, pl.ds(j * 16, 16)]
                )
            return 0
        lax.fori_loop(0, TCH, tok, 0)
        pltpu.sync_copy(r0, out_hbm.at[pl.ds(wbase + c * TCH, TCH)])

    gather(0, a0, a1, sa)

    def chunk(c, _):
        @pl.when(c % 2 == 0)
        def _():
            @pl.when(c + 1 < NCH)
            def _():
                gather(c + 1, b0, b1, sb)
            compute(c, a0, a1, sa)

        @pl.when(c % 2 == 1)
        def _():
            @pl.when(c + 1 < NCH)
            def _():
                gather(c + 1, a0, a1, sa)
            compute(c, b0, b1, sb)
        return 0
    lax.fori_loop(0, NCH, chunk, 0)


def kernel(x, y, W_experts, b_experts, gate_W, gate_b):
    Wb = W_experts.astype(jnp.bfloat16)
    b3 = b_experts.reshape(E, 1, D)
    gb2 = gate_b.reshape(1, E)

    lg = pl.pallas_call(
        _gate_body,
        grid=(B // BT,),
        in_specs=[
            pl.BlockSpec((BT, DH), lambda t: (t, 0)),
            pl.BlockSpec((BT, DH), lambda t: (t, 0)),
            pl.BlockSpec((E, D), lambda t: (0, 0)),
            pl.BlockSpec((1, E), lambda t: (0, 0)),
        ],
        out_specs=pl.BlockSpec((BT, E), lambda t: (t, 0)),
        out_shape=jax.ShapeDtypeStruct((B, E), jnp.float32),
    )(x, y, gate_W, gb2)

    p0, p1, g0, g1, te = pl.pallas_call(
        _route_body,
        out_shape=[
            jax.ShapeDtypeStruct((B, 1), jnp.int32),
            jax.ShapeDtypeStruct((B, 1), jnp.int32),
            jax.ShapeDtypeStruct((B, 16), jnp.float32),
            jax.ShapeDtypeStruct((B, 16), jnp.float32),
            jax.ShapeDtypeStruct((8, 128), jnp.int32),
        ],
    )(lg)
    p0f = p0.reshape(B)
    p1f = p1.reshape(B)
    te_row = te[0]

    mesh = plsc.VectorSubcoreMesh(core_axis_name="c", subcore_axis_name="s")
    xa, xc = pl.kernel(
        _dispatch_body,
        out_type=[
            jax.ShapeDtypeStruct((M_PAD, DH), jnp.float32),
            jax.ShapeDtypeStruct((M_PAD, DH), jnp.float32),
        ],
        mesh=mesh,
        scratch_types=[
            pltpu.VMEM((16,), jnp.int32),
            pltpu.VMEM((16,), jnp.int32),
            pltpu.VMEM((16, DH), jnp.float32),
            pltpu.VMEM((16, DH), jnp.float32),
            pltpu.VMEM((16, DH), jnp.float32),
            pltpu.VMEM((16, DH), jnp.float32),
            pltpu.SemaphoreType.DMA,
            pltpu.SemaphoreType.DMA,
            pltpu.SemaphoreType.DMA,
        ],
    )(x, y, p0f, p1f)

    ysort = pl.pallas_call(
        _gmm_body,
        grid_spec=pltpu.PrefetchScalarGridSpec(
            num_scalar_prefetch=1,
            grid=(M_TILES,),
            in_specs=[
                pl.BlockSpec((BM, DH), lambda m, te_r: (m, 0)),
                pl.BlockSpec((BM, DH), lambda m, te_r: (m, 0)),
                pl.BlockSpec((1, BN, D), lambda m, te_r: (te_r[m], 0, 0)),
                pl.BlockSpec((1, 1, BN), lambda m, te_r: (te_r[m], 0, 0)),
            ],
            out_specs=pl.BlockSpec((BM, BN), lambda m, te_r: (m, 0)),
        ),
        out_shape=jax.ShapeDtypeStruct((M_PAD, D), jnp.float32),
        compiler_params=pltpu.CompilerParams(
            dimension_semantics=("arbitrary",),
        ),
    )(te_row, xa, xc, Wb, b3)

    out = pl.kernel(
        _combine_body,
        out_type=jax.ShapeDtypeStruct((B, D), jnp.float32),
        mesh=mesh,
        scratch_types=[
            pltpu.VMEM((TPW,), jnp.int32),
            pltpu.VMEM((TPW,), jnp.int32),
            pltpu.VMEM((TPW, 16), jnp.float32),
            pltpu.VMEM((8, D), jnp.float32),
            pltpu.VMEM((8, D), jnp.float32),
            pltpu.VMEM((8, D), jnp.float32),
            pltpu.VMEM((8, D), jnp.float32),
            pltpu.SemaphoreType.DMA,
            pltpu.SemaphoreType.DMA,
        ],
    )(ysort, p0f, p1f, g0)
    return out


# speedup vs baseline: 3.3764x; 1.0023x over previous
import functools

import jax
import jax.numpy as jnp
from jax import lax
from jax.experimental import pallas as pl
from jax.experimental.pallas import tpu as pltpu
from jax.experimental.pallas import tpu_sc as plsc

B = 8192
DH = 1024
D = 2048
E = 8
BT = 512
BM = 256
BN = 2048
M_PAD = B * 2 + E * BM
M_TILES = M_PAD // BM
N_TILES = D // BN
NW = 32
TPW = B // NW


def _gate_body(x_ref, y_ref, gw_ref, gb_ref, lg_ref):
    xv = x_ref[...]
    yv = y_ref[...]
    dnums = (((1,), (1,)), ((), ()))
    inpb = jnp.concatenate([xv, yv], axis=1).astype(jnp.bfloat16)
    gwb = gw_ref[...].astype(jnp.bfloat16)
    lg_ref[...] = (
        lax.dot_general(inpb, gwb, dnums, preferred_element_type=jnp.float32)
        + gb_ref[...]
    )


def _route_body(lg_ref, p0_ref, p1_ref, g0_ref, g1_ref, te_ref):
    lg = lg_ref[...]
    idx8 = lax.broadcasted_iota(jnp.int32, (B, E), 1)
    m1 = jnp.max(lg, axis=1, keepdims=True)
    i1 = jnp.min(jnp.where(lg == m1, idx8, E), axis=1, keepdims=True)
    l2 = jnp.where(idx8 == i1, -1e30, lg)
    m2 = jnp.max(l2, axis=1, keepdims=True)
    i2 = jnp.min(jnp.where(l2 == m2, idx8, E), axis=1, keepdims=True)
    g0 = 1.0 / (1.0 + jnp.exp(m2 - m1))
    g1 = 1.0 - g0
    oh1 = (idx8 == i1).astype(jnp.float32)
    oh2 = (idx8 == i2).astype(jnp.float32)
    h = oh1 + oh2

    CH = 512
    tri = (
        lax.broadcasted_iota(jnp.int32, (CH, CH), 0)
        > lax.broadcasted_iota(jnp.int32, (CH, CH), 1)
    ).astype(jnp.float32)
    dn = (((1,), (0,)), ((), ()))
    carry = jnp.zeros((1, E), jnp.float32)
    chunks = []
    for c in range(B // CH):
        hc = lax.slice(h, (c * CH, 0), ((c + 1) * CH, E))
        chunks.append(carry + lax.dot_general(tri, hc, dn,
                                              preferred_element_type=jnp.float32))
        carry = carry + jnp.sum(hc, axis=0, keepdims=True)
    hexc = jnp.concatenate(chunks, axis=0)
    counts = carry.astype(jnp.int32)

    ntiles = (counts + (BM - 1)) // BM
    tri8 = (
        lax.broadcasted_iota(jnp.int32, (E, E), 0)
        < lax.broadcasted_iota(jnp.int32, (E, E), 1)
    ).astype(jnp.float32)
    off_t = lax.dot_general(ntiles.astype(jnp.float32), tri8, dn,
                            preferred_element_type=jnp.float32)
    off_rows = off_t * BM

    posf = hexc + off_rows
    p0_ref[...] = jnp.sum(oh1 * posf, axis=1, keepdims=True).astype(jnp.int32)
    p1_ref[...] = jnp.sum(oh2 * posf, axis=1, keepdims=True).astype(jnp.int32)
    g0_ref[...] = jnp.broadcast_to(g0, (B, 16))
    g1_ref[...] = jnp.broadcast_to(g1, (B, 16))

    off_ti = off_t.astype(jnp.int32)
    mti = lax.broadcasted_iota(jnp.int32, (1, 128), 1)
    te = jnp.zeros((1, 128), jnp.int32)
    for e in range(E):
        te = te + (mti >= lax.slice(off_ti, (0, e), (1, e + 1))).astype(jnp.int32)
    te = jnp.clip(te - 1, 0, E - 1)
    te_ref[...] = jnp.broadcast_to(te, (8, 128))


def _dispatch_body(xb_hbm, yb_hbm, p0_hbm, p1_hbm, xa_hbm, xc_hbm,
                   i0_v, i1_v, xva, yva, xvb, yvb, sla, slb, ssc):
    wid = lax.axis_index("s") * 2 + lax.axis_index("c")
    CH = 16
    NCH = TPW // CH

    def load(c, xv, yv, sem):
        base = wid * TPW + c * CH
        pltpu.async_copy(xb_hbm.at[pl.ds(base, CH)], xv, sem)
        pltpu.async_copy(yb_hbm.at[pl.ds(base, CH)], yv, sem)

    def scat(c, xv, yv, sem):
        pltpu.async_copy(xv, xa_hbm.at[i0_v.at[c]], sem)
        pltpu.async_copy(xv, xa_hbm.at[i1_v.at[c]], sem)
        pltpu.async_copy(yv, xc_hbm.at[i0_v.at[c]], sem)
        cp = pltpu.async_copy(yv, xc_hbm.at[i1_v.at[c]], sem)
        cp.wait()
        cp.wait()
        cp.wait()
        cp.wait()

    pltpu.sync_copy(p0_hbm.at[wid], i0_v)
    pltpu.sync_copy(p1_hbm.at[wid], i1_v)
    load(0, xva, yva, sla)

    def chunk(c, _):
        @pl.when(c % 2 == 0)
        def _():
            cp = pltpu.make_async_copy(xb_hbm.at[pl.ds(0, CH)], xva, sla)
            cp.wait()
            cp.wait()
            @pl.when(c + 1 < NCH)
            def _():
                load(c + 1, xvb, yvb, slb)
            scat(c, xva, yva, ssc)

        @pl.when(c % 2 == 1)
        def _():
            cp = pltpu.make_async_copy(xb_hbm.at[pl.ds(0, CH)], xvb, slb)
            cp.wait()
            cp.wait()
            @pl.when(c + 1 < NCH)
            def _():
                load(c + 1, xva, yva, sla)
            scat(c, xvb, yvb, ssc)
        return 0
    lax.fori_loop(0, NCH, chunk, 0)


def _gmm_body(te_ref, xa_ref, xc_ref, w_ref, b_ref, y_ref):
    dnums = (((1,), (1,)), ((), ()))
    wa = w_ref[0, :, :DH]
    wb = w_ref[0, :, DH:]
    acc = (
        lax.dot_general(xa_ref[...].astype(jnp.bfloat16), wa, dnums,
                        preferred_element_type=jnp.float32)
        + lax.dot_general(xc_ref[...].astype(jnp.bfloat16), wb, dnums,
                          preferred_element_type=jnp.float32)
    )
    y_ref[...] = acc + b_ref[0]


def _combine_body(y_hbm, p0_hbm, p1_hbm, g0_hbm, out_hbm,
                  i0_v, i1_v, gg0, a0, a1, b0, b1, sa, sb):
    wid = lax.axis_index("s") * 2 + lax.axis_index("c")
    wbase = wid * TPW
    pltpu.sync_copy(p0_hbm.at[pl.ds(wbase, TPW)], i0_v)
    pltpu.sync_copy(p1_hbm.at[pl.ds(wbase, TPW)], i1_v)
    pltpu.sync_copy(g0_hbm.at[pl.ds(wbase, TPW)], gg0)
    TCH = 8
    NCH = TPW // TCH

    def gather(c, r0, r1, sem):
        pltpu.async_copy(y_hbm.at[i0_v.at[pl.ds(c * TCH, TCH)]], r0, sem)
        pltpu.async_copy(y_hbm.at[i1_v.at[pl.ds(c * TCH, TCH)]], r1, sem)

    def compute(c, r0, r1, sem):
        cp = pltpu.make_async_copy(y_hbm.at[pl.ds(0, TCH)], r0, sem)
        cp.wait()
        cp.wait()

        def tok(i, _):
            g0s = gg0[c * TCH + i, :]
            g1s = 1.0 - g0s
            for j in range(D // 16):
                r0[i, pl.ds(j * 16, 16)] = (
                    g0s * r0[i, pl.ds(j * 16, 16)]
                    + g1s * r1[i, pl.ds(j * 16, 16)]
                )
            return 0
        lax.fori_loop(0, TCH, tok, 0)
        pltpu.sync_copy(r0, out_hbm.at[pl.ds(wbase + c * TCH, TCH)])

    gather(0, a0, a1, sa)

    def chunk(c, _):
        @pl.when(c % 2 == 0)
        def _():
            @pl.when(c + 1 < NCH)
            def _():
                gather(c + 1, b0, b1, sb)
            compute(c, a0, a1, sa)

        @pl.when(c % 2 == 1)
        def _():
            @pl.when(c + 1 < NCH)
            def _():
                gather(c + 1, a0, a1, sa)
            compute(c, b0, b1, sb)
        return 0
    lax.fori_loop(0, NCH, chunk, 0)


def kernel(x, y, W_experts, b_experts, gate_W, gate_b):
    Wb = W_experts.astype(jnp.bfloat16)
    b3 = b_experts.reshape(E, 1, D)
    gb2 = gate_b.reshape(1, E)

    lg = pl.pallas_call(
        _gate_body,
        grid=(B // BT,),
        in_specs=[
            pl.BlockSpec((BT, DH), lambda t: (t, 0)),
            pl.BlockSpec((BT, DH), lambda t: (t, 0)),
            pl.BlockSpec((E, D), lambda t: (0, 0)),
            pl.BlockSpec((1, E), lambda t: (0, 0)),
        ],
        out_specs=pl.BlockSpec((BT, E), lambda t: (t, 0)),
        out_shape=jax.ShapeDtypeStruct((B, E), jnp.float32),
    )(x, y, gate_W, gb2)

    p0, p1, g0, g1, te = pl.pallas_call(
        _route_body,
        out_shape=[
            jax.ShapeDtypeStruct((B, 1), jnp.int32),
            jax.ShapeDtypeStruct((B, 1), jnp.int32),
            jax.ShapeDtypeStruct((B, 16), jnp.float32),
            jax.ShapeDtypeStruct((B, 16), jnp.float32),
            jax.ShapeDtypeStruct((8, 128), jnp.int32),
        ],
    )(lg)
    p0f = p0.reshape(B)
    p1f = p1.reshape(B)
    te_row = te[0]

    mesh = plsc.VectorSubcoreMesh(core_axis_name="c", subcore_axis_name="s")
    xa, xc = pl.kernel(
        _dispatch_body,
        out_type=[
            jax.ShapeDtypeStruct((M_PAD, DH), jnp.float32),
            jax.ShapeDtypeStruct((M_PAD, DH), jnp.float32),
        ],
        mesh=mesh,
        scratch_types=[
            pltpu.VMEM((16, 16), jnp.int32),
            pltpu.VMEM((16, 16), jnp.int32),
            pltpu.VMEM((16, DH), jnp.float32),
            pltpu.VMEM((16, DH), jnp.float32),
            pltpu.VMEM((16, DH), jnp.float32),
            pltpu.VMEM((16, DH), jnp.float32),
            pltpu.SemaphoreType.DMA,
            pltpu.SemaphoreType.DMA,
            pltpu.SemaphoreType.DMA,
        ],
    )(x, y, p0f.reshape(NW, 16, 16), p1f.reshape(NW, 16, 16))

    ysort = pl.pallas_call(
        _gmm_body,
        grid_spec=pltpu.PrefetchScalarGridSpec(
            num_scalar_prefetch=1,
            grid=(M_TILES,),
            in_specs=[
                pl.BlockSpec((BM, DH), lambda m, te_r: (m, 0)),
                pl.BlockSpec((BM, DH), lambda m, te_r: (m, 0)),
                pl.BlockSpec((1, BN, D), lambda m, te_r: (te_r[m], 0, 0)),
                pl.BlockSpec((1, 1, BN), lambda m, te_r: (te_r[m], 0, 0)),
            ],
            out_specs=pl.BlockSpec((BM, BN), lambda m, te_r: (m, 0)),
        ),
        out_shape=jax.ShapeDtypeStruct((M_PAD, D), jnp.float32),
        compiler_params=pltpu.CompilerParams(
            dimension_semantics=("arbitrary",),
        ),
    )(te_row, xa, xc, Wb, b3)

    out = pl.kernel(
        _combine_body,
        out_type=jax.ShapeDtypeStruct((B, D), jnp.float32),
        mesh=mesh,
        scratch_types=[
            pltpu.VMEM((TPW,), jnp.int32),
            pltpu.VMEM((TPW,), jnp.int32),
            pltpu.VMEM((TPW, 16), jnp.float32),
            pltpu.VMEM((8, D), jnp.float32),
            pltpu.VMEM((8, D), jnp.float32),
            pltpu.VMEM((8, D), jnp.float32),
            pltpu.VMEM((8, D), jnp.float32),
            pltpu.SemaphoreType.DMA,
            pltpu.SemaphoreType.DMA,
        ],
    )(ysort, p0f, p1f, g0)
    return out
